# baseline (speedup 1.0000x reference)
import math

import jax
import jax.numpy as jnp
from jax.experimental import pallas as pl

_NBINS = 10
_POOL = 8
_H = 512
_W = 512
_HP = _H // _POOL
_WP = _W // _POOL


def _hog_body(x_ref, wt_ref, wv_ref, p1_ref, p2_ref, o_ref):
    xb = x_ref[0].astype(jnp.bfloat16)

    t = jax.lax.dot_general(
        wt_ref[...], xb, (((1,), (0,)), ((), ())),
        preferred_element_type=jnp.float32)
    v = jax.lax.dot_general(
        wv_ref[...], xb, (((1,), (0,)), ((), ())),
        preferred_element_type=jnp.float32)

    zcol = jnp.zeros((_H, 1), jnp.float32)
    t_l = jnp.concatenate([zcol, t[:, :-1]], axis=1)
    t_r = jnp.concatenate([t[:, 1:], zcol], axis=1)
    g0 = t_l - t_r
    v_l = jnp.concatenate([zcol, v[:, :-1]], axis=1)
    v_r = jnp.concatenate([v[:, 1:], zcol], axis=1)
    g1 = v_l + 2.0 * v + v_r

    mag = jnp.sqrt(g0 * g0 + g1 * g1)

    zero0 = g0 == 0.0
    edge = zero0 | (g1 == 0.0)
    w_a = jnp.where(edge, 1.0, mag)
    w_b = jnp.where(edge, 0.0, 1.0 - mag)

    a = jnp.abs(g0)
    b = jnp.where(g0 < 0.0, -g1, g1)
    r = jnp.where(zero0, jnp.inf, b / a)

    p1 = p1_ref[...]
    p2 = p2_ref[...]

    def _pool(arr):
        rp = jax.lax.dot_general(
            p1, arr, (((1,), (0,)), ((), ())),
            preferred_element_type=jnp.float32)
        return jax.lax.dot_general(
            rp, p2, (((1,), (0,)), ((), ())),
            preferred_element_type=jnp.float32)

    acc_a = [_pool(w_a)]
    acc_b = [_pool(w_b)]
    for j in range(1, _NBINS):
        phi = j * math.pi / _NBINS
        cot = math.cos(phi) / math.sin(phi)
        m = r <= cot
        acc_a.append(_pool(jnp.where(m, w_a, 0.0)))
        acc_b.append(_pool(jnp.where(m, w_b, 0.0)))

    scale = 1.0 / (_POOL * _POOL)
    for k in range(_NBINS):
        a_hi = acc_a[k + 1] if k + 1 < _NBINS else 0.0
        b_lo = acc_b[k - 1] if k >= 1 else acc_b[_NBINS - 1]
        b_hi = acc_b[k] if k >= 1 else 0.0
        o_ref[0, k] = (acc_a[k] - a_hi + b_lo - b_hi) * scale


def kernel(x):
    n = x.shape[0]
    x2 = x.reshape(n, _H, _W)

    rows = jax.lax.broadcasted_iota(jnp.int32, (_H, _H), 0)
    cols = jax.lax.broadcasted_iota(jnp.int32, (_H, _H), 1)
    diff = rows - cols
    w_t = (jnp.where(diff == 0, 2.0, 0.0)
           + jnp.where(jnp.abs(diff) == 1, 1.0, 0.0)).astype(jnp.bfloat16)
    w_v = (jnp.where(diff == 1, 1.0, 0.0)
           - jnp.where(diff == -1, 1.0, 0.0)).astype(jnp.bfloat16)

    pr = jax.lax.broadcasted_iota(jnp.int32, (_HP, _H), 0)
    pc = jax.lax.broadcasted_iota(jnp.int32, (_HP, _H), 1)
    p1 = jnp.where(pr == pc // _POOL, 1.0, 0.0).astype(jnp.float32)
    p2 = p1.T

    out = pl.pallas_call(
        _hog_body,
        grid=(n,),
        in_specs=[
            pl.BlockSpec((1, _H, _W), lambda bidx: (bidx, 0, 0)),
            pl.BlockSpec((_H, _H), lambda bidx: (0, 0)),
            pl.BlockSpec((_H, _H), lambda bidx: (0, 0)),
            pl.BlockSpec((_HP, _H), lambda bidx: (0, 0)),
            pl.BlockSpec((_W, _WP), lambda bidx: (0, 0)),
        ],
        out_specs=pl.BlockSpec(
            (1, _NBINS, _HP, _WP), lambda bidx: (bidx, 0, 0, 0)),
        out_shape=jax.ShapeDtypeStruct((n, _NBINS, _HP, _WP), jnp.float32),
    )(x2, w_t, w_v, p1, p2)
    return out


# speedup vs baseline: 1.1651x; 1.1651x over previous
import math

import jax
import jax.numpy as jnp
from jax.experimental import pallas as pl

_NBINS = 10
_POOL = 8
_H = 512
_W = 512
_HP = _H // _POOL
_WP = _W // _POOL


def _hog_body(x_ref, p1_ref, p1b_ref, p2_ref, o_ref):
    x = x_ref[0].astype(jnp.bfloat16).astype(jnp.float32)

    zrow = jnp.zeros((1, _W), jnp.float32)
    x_up = jnp.concatenate([zrow, x[:-1, :]], axis=0)
    x_dn = jnp.concatenate([x[1:, :], zrow], axis=0)
    t = x_up + 2.0 * x + x_dn
    v = x_up - x_dn

    zcol = jnp.zeros((_H, 1), jnp.float32)
    t_l = jnp.concatenate([zcol, t[:, :-1]], axis=1)
    t_r = jnp.concatenate([t[:, 1:], zcol], axis=1)
    g0 = t_l - t_r
    v_l = jnp.concatenate([zcol, v[:, :-1]], axis=1)
    v_r = jnp.concatenate([v[:, 1:], zcol], axis=1)
    g1 = v_l + 2.0 * v + v_r

    mag = jnp.sqrt(g0 * g0 + g1 * g1)

    zero0 = g0 == 0.0
    edge = zero0 | (g1 == 0.0)
    w_a = jnp.where(edge, 1.0, mag)

    a = jnp.abs(g0)
    b = jnp.where(g0 < 0.0, -g1, g1)
    r = jnp.where(zero0, jnp.inf, b / a)

    p1 = p1_ref[...]
    p1b = p1b_ref[...]
    p2 = p2_ref[...]

    def _pool(arr):
        rp = jax.lax.dot_general(
            p1, arr, (((1,), (0,)), ((), ())),
            preferred_element_type=jnp.float32)
        return jax.lax.dot_general(
            rp, p2, (((1,), (0,)), ((), ())),
            preferred_element_type=jnp.float32)

    def _pool_mask(arr_bf16):
        rp = jax.lax.dot_general(
            p1b, arr_bf16, (((1,), (0,)), ((), ())),
            preferred_element_type=jnp.float32)
        return jax.lax.dot_general(
            rp, p2, (((1,), (0,)), ((), ())),
            preferred_element_type=jnp.float32)

    acc_a = [_pool(w_a)]
    acc_s = [jnp.float32(_POOL * _POOL)]
    for j in range(1, _NBINS):
        phi = j * math.pi / _NBINS
        cot = math.cos(phi) / math.sin(phi)
        m = r <= cot
        acc_a.append(_pool(jnp.where(m, w_a, 0.0)))
        acc_s.append(_pool_mask(
            jnp.where(m, 1.0, 0.0).astype(jnp.bfloat16)))

    scale = 1.0 / (_POOL * _POOL)
    for k in range(_NBINS):
        a_hi = acc_a[k + 1] if k + 1 < _NBINS else 0.0
        if k >= 1:
            b_part = (acc_s[k - 1] - acc_a[k - 1]) - (acc_s[k] - acc_a[k])
        else:
            b_part = acc_s[_NBINS - 1] - acc_a[_NBINS - 1]
        o_ref[0, k] = (acc_a[k] - a_hi + b_part) * scale


def kernel(x):
    n = x.shape[0]
    x2 = x.reshape(n, _H, _W)

    pr = jax.lax.broadcasted_iota(jnp.int32, (_HP, _H), 0)
    pc = jax.lax.broadcasted_iota(jnp.int32, (_HP, _H), 1)
    p1 = jnp.where(pr == pc // _POOL, 1.0, 0.0).astype(jnp.float32)
    p1b = p1.astype(jnp.bfloat16)
    p2 = p1.T

    out = pl.pallas_call(
        _hog_body,
        grid=(n,),
        in_specs=[
            pl.BlockSpec((1, _H, _W), lambda bidx: (bidx, 0, 0)),
            pl.BlockSpec((_HP, _H), lambda bidx: (0, 0)),
            pl.BlockSpec((_HP, _H), lambda bidx: (0, 0)),
            pl.BlockSpec((_W, _WP), lambda bidx: (0, 0)),
        ],
        out_specs=pl.BlockSpec(
            (1, _NBINS, _HP, _WP), lambda bidx: (bidx, 0, 0, 0)),
        out_shape=jax.ShapeDtypeStruct((n, _NBINS, _HP, _WP), jnp.float32),
    )(x2, p1, p1b, p2)
    return out


# speedup vs baseline: 1.8100x; 1.5535x over previous
import math

import jax
import jax.numpy as jnp
from jax.experimental import pallas as pl

_NBINS = 10
_POOL = 8
_H = 512
_W = 512
_HP = _H // _POOL
_WP = _W // _POOL
_IMGS_PER_STEP = 2


def _hog_body(x_ref, p1_ref, p1b_ref, p2_ref, o_ref):
    for img in range(_IMGS_PER_STEP):
        _hog_one(x_ref, p1_ref, p1b_ref, p2_ref, o_ref, img)


def _hog_one(x_ref, p1_ref, p1b_ref, p2_ref, o_ref, img):
    x = x_ref[img].astype(jnp.bfloat16).astype(jnp.float32)

    zrow = jnp.zeros((1, _W), jnp.float32)
    x_up = jnp.concatenate([zrow, x[:-1, :]], axis=0)
    x_dn = jnp.concatenate([x[1:, :], zrow], axis=0)
    t = x_up + 2.0 * x + x_dn
    v = x_up - x_dn

    zcol = jnp.zeros((_H, 1), jnp.float32)
    t_l = jnp.concatenate([zcol, t[:, :-1]], axis=1)
    t_r = jnp.concatenate([t[:, 1:], zcol], axis=1)
    g0 = t_l - t_r
    v_l = jnp.concatenate([zcol, v[:, :-1]], axis=1)
    v_r = jnp.concatenate([v[:, 1:], zcol], axis=1)
    g1 = v_l + 2.0 * v + v_r

    mag = jnp.sqrt(g0 * g0 + g1 * g1)

    zero0 = g0 == 0.0
    edge = zero0 | (g1 == 0.0)
    w_a = jnp.where(edge, 1.0, mag)

    a = jnp.abs(g0)
    b = jnp.where(g0 < 0.0, -g1, g1)
    r = jnp.where(zero0, jnp.inf, b / a)

    p1 = p1_ref[...]
    p1b = p1b_ref[...]
    p2 = p2_ref[...]

    sel_a = [w_a]
    sel_s = []
    for j in range(1, _NBINS):
        phi = j * math.pi / _NBINS
        cot = math.cos(phi) / math.sin(phi)
        m = r <= cot
        sel_a.append(jnp.where(m, w_a, 0.0))
        sel_s.append(jnp.where(m, 1.0, 0.0).astype(jnp.bfloat16))

    def _rowpool(arr):
        return jax.lax.dot_general(
            p1, arr, (((1,), (0,)), ((), ())),
            preferred_element_type=jnp.float32)

    def _rowpool_b(arr):
        return jax.lax.dot_general(
            p1b, arr, (((1,), (0,)), ((), ())),
            preferred_element_type=jnp.float32)

    rp_a = [_rowpool(arr) for arr in sel_a]
    rp_s = [_rowpool_b(arr) for arr in sel_s]

    def _colpool(rp):
        return jax.lax.dot_general(
            rp, p2, (((1,), (0,)), ((), ())),
            preferred_element_type=jnp.float32)

    acc_a = [_colpool(rp) for rp in rp_a]
    acc_s = [jnp.float32(_POOL * _POOL)] + [_colpool(rp) for rp in rp_s]

    scale = 1.0 / (_POOL * _POOL)
    for k in range(_NBINS):
        a_hi = acc_a[k + 1] if k + 1 < _NBINS else 0.0
        if k >= 1:
            b_part = (acc_s[k - 1] - acc_a[k - 1]) - (acc_s[k] - acc_a[k])
        else:
            b_part = acc_s[_NBINS - 1] - acc_a[_NBINS - 1]
        o_ref[img, k] = (acc_a[k] - a_hi + b_part) * scale


def kernel(x):
    n = x.shape[0]
    x2 = x.reshape(n, _H, _W)

    pr = jax.lax.broadcasted_iota(jnp.int32, (_HP, _H), 0)
    pc = jax.lax.broadcasted_iota(jnp.int32, (_HP, _H), 1)
    p1 = jnp.where(pr == pc // _POOL, 1.0, 0.0).astype(jnp.float32)
    p1b = p1.astype(jnp.bfloat16)
    p2 = p1.T

    steps = n // _IMGS_PER_STEP
    out = pl.pallas_call(
        _hog_body,
        grid=(steps,),
        in_specs=[
            pl.BlockSpec((_IMGS_PER_STEP, _H, _W), lambda bidx: (bidx, 0, 0)),
            pl.BlockSpec((_HP, _H), lambda bidx: (0, 0)),
            pl.BlockSpec((_HP, _H), lambda bidx: (0, 0)),
            pl.BlockSpec((_W, _WP), lambda bidx: (0, 0)),
        ],
        out_specs=pl.BlockSpec(
            (_IMGS_PER_STEP, _NBINS, _HP, _WP),
            lambda bidx: (bidx, 0, 0, 0)),
        out_shape=jax.ShapeDtypeStruct((n, _NBINS, _HP, _WP), jnp.float32),
    )(x2, p1, p1b, p2)
    return out


# speedup vs baseline: 1.8677x; 1.0319x over previous
import math

import jax
import jax.numpy as jnp
from jax.experimental import pallas as pl

_NBINS = 10
_POOL = 8
_H = 512
_W = 512
_HP = _H // _POOL
_WP = _W // _POOL
_IMGS_PER_STEP = 4


def _hog_body(x_ref, p1_ref, p1b_ref, p2_ref, o_ref):
    for img in range(_IMGS_PER_STEP):
        _hog_one(x_ref, p1_ref, p1b_ref, p2_ref, o_ref, img)


def _hog_one(x_ref, p1_ref, p1b_ref, p2_ref, o_ref, img):
    x = x_ref[img].astype(jnp.bfloat16).astype(jnp.float32)

    zrow = jnp.zeros((1, _W), jnp.float32)
    x_up = jnp.concatenate([zrow, x[:-1, :]], axis=0)
    x_dn = jnp.concatenate([x[1:, :], zrow], axis=0)
    t = x_up + 2.0 * x + x_dn
    v = x_up - x_dn

    zcol = jnp.zeros((_H, 1), jnp.float32)
    t_l = jnp.concatenate([zcol, t[:, :-1]], axis=1)
    t_r = jnp.concatenate([t[:, 1:], zcol], axis=1)
    g0 = t_l - t_r
    v_l = jnp.concatenate([zcol, v[:, :-1]], axis=1)
    v_r = jnp.concatenate([v[:, 1:], zcol], axis=1)
    g1 = v_l + 2.0 * v + v_r

    mag = jnp.sqrt(g0 * g0 + g1 * g1)

    zero0 = g0 == 0.0
    edge = zero0 | (g1 == 0.0)
    w_a = jnp.where(edge, 1.0, mag)

    a = jnp.abs(g0)
    b = jnp.where(g0 < 0.0, -g1, g1)
    r = jnp.where(zero0, jnp.inf, b / a)

    p1 = p1_ref[...]
    p1b = p1b_ref[...]
    p2 = p2_ref[...]

    sel_a = [w_a]
    sel_s = []
    for j in range(1, _NBINS):
        phi = j * math.pi / _NBINS
        cot = math.cos(phi) / math.sin(phi)
        m = r <= cot
        sel_a.append(jnp.where(m, w_a, 0.0))
        sel_s.append(jnp.where(m, 1.0, 0.0).astype(jnp.bfloat16))

    def _rowpool(arr):
        return jax.lax.dot_general(
            p1, arr, (((1,), (0,)), ((), ())),
            preferred_element_type=jnp.float32)

    def _rowpool_b(arr):
        return jax.lax.dot_general(
            p1b, arr, (((1,), (0,)), ((), ())),
            preferred_element_type=jnp.float32)

    rp_a = [_rowpool(arr) for arr in sel_a]
    rp_s = [_rowpool_b(arr) for arr in sel_s]

    def _colpool(rp):
        return jax.lax.dot_general(
            rp, p2, (((1,), (0,)), ((), ())),
            preferred_element_type=jnp.float32)

    acc_a = [_colpool(rp) for rp in rp_a]
    acc_s = [jnp.float32(_POOL * _POOL)] + [_colpool(rp) for rp in rp_s]

    scale = 1.0 / (_POOL * _POOL)
    for k in range(_NBINS):
        a_hi = acc_a[k + 1] if k + 1 < _NBINS else 0.0
        if k >= 1:
            b_part = (acc_s[k - 1] - acc_a[k - 1]) - (acc_s[k] - acc_a[k])
        else:
            b_part = acc_s[_NBINS - 1] - acc_a[_NBINS - 1]
        o_ref[img, k] = (acc_a[k] - a_hi + b_part) * scale


def kernel(x):
    n = x.shape[0]
    x2 = x.reshape(n, _H, _W)

    pr = jax.lax.broadcasted_iota(jnp.int32, (_HP, _H), 0)
    pc = jax.lax.broadcasted_iota(jnp.int32, (_HP, _H), 1)
    p1 = jnp.where(pr == pc // _POOL, 1.0, 0.0).astype(jnp.float32)
    p1b = p1.astype(jnp.bfloat16)
    p2 = p1.T

    steps = n // _IMGS_PER_STEP
    out = pl.pallas_call(
        _hog_body,
        grid=(steps,),
        in_specs=[
            pl.BlockSpec((_IMGS_PER_STEP, _H, _W), lambda bidx: (bidx, 0, 0)),
            pl.BlockSpec((_HP, _H), lambda bidx: (0, 0)),
            pl.BlockSpec((_HP, _H), lambda bidx: (0, 0)),
            pl.BlockSpec((_W, _WP), lambda bidx: (0, 0)),
        ],
        out_specs=pl.BlockSpec(
            (_IMGS_PER_STEP, _NBINS, _HP, _WP),
            lambda bidx: (bidx, 0, 0, 0)),
        out_shape=jax.ShapeDtypeStruct((n, _NBINS, _HP, _WP), jnp.float32),
    )(x2, p1, p1b, p2)
    return out
